# baseline (speedup 1.0000x reference)
import functools

import jax
import jax.numpy as jnp
from jax import lax
from jax.experimental import pallas as pl
from jax.experimental.pallas import tpu as pltpu
from jax.experimental.pallas import tpu_sc as plsc

DIM = 64
PDIM = 128
BATCH = 16384
HIST = 50
B = BATCH * HIST

_info = plsc.get_sparse_core_info()
NC = _info.num_cores
NS = _info.num_subcores
NW = NC * NS
ROWS_PER_W = BATCH // NW
B_PER_W = ROWS_PER_W * HIST
CHUNK = 128
N_CHUNKS = B_PER_W // CHUNK
NBUF = 4
N_OUTER = N_CHUNKS // NBUF
L = 16

_mesh = plsc.VectorSubcoreMesh(core_axis_name="c", subcore_axis_name="s")


@functools.partial(
    pl.kernel,
    mesh=_mesh,
    out_type=jax.ShapeDtypeStruct((B, PDIM), jnp.float32),
    scratch_types=[
        pltpu.VMEM((ROWS_PER_W, HIST), jnp.int32),
        pltpu.VMEM((B_PER_W,), jnp.int32),
        *[pltpu.VMEM((CHUNK, PDIM), jnp.float32) for _ in range(NBUF)],
        *[pltpu.SemaphoreType.DMA for _ in range(2 * NBUF)],
    ],
    compiler_params=pltpu.CompilerParams(use_tc_tiling_on_sc=False),
)
def _gather(idx_hbm, table_hbm, out_hbm, idx2d, idx_flat,
            r0, r1, r2, r3, g0, g1, g2, g3, s0, s1, s2, s3):
    rows = (r0, r1, r2, r3)
    semG = (g0, g1, g2, g3)
    semS = (s0, s1, s2, s3)
    wid = lax.axis_index("s") * NC + lax.axis_index("c")
    base = wid * B_PER_W

    pltpu.sync_copy(idx_hbm.at[pl.ds(wid * ROWS_PER_W, ROWS_PER_W)], idx2d)

    @pl.loop(0, ROWS_PER_W)
    def _flatten(r):
        off = r * HIST
        idx_flat[pl.ds(off, L)] = idx2d[r, pl.ds(0, L)]
        idx_flat[pl.ds(off + L, L)] = idx2d[r, pl.ds(L, L)]
        idx_flat[pl.ds(off + 2 * L, L)] = idx2d[r, pl.ds(2 * L, L)]
        idx_flat[pl.ds(off + HIST - L, L)] = idx2d[r, pl.ds(HIST - L, L)]

    def start_gather(i, b):
        pltpu.async_copy(
            table_hbm.at[idx_flat.at[pl.ds(i * CHUNK, CHUNK)]], rows[b], semG[b])

    def wait_gather(i, b):
        pltpu.make_async_copy(
            table_hbm.at[idx_flat.at[pl.ds(i * CHUNK, CHUNK)]], rows[b], semG[b]).wait()

    def start_store(i, b):
        pltpu.async_copy(
            rows[b].at[:, pl.ds(0, DIM)],
            out_hbm.at[pl.ds(base + i * CHUNK, CHUNK), pl.ds(0, DIM)], semS[b])

    def wait_store(i, b):
        pltpu.make_async_copy(
            rows[b].at[:, pl.ds(0, DIM)],
            out_hbm.at[pl.ds(base + i * CHUNK, CHUNK), pl.ds(0, DIM)], semS[b]).wait()

    for b in range(NBUF):
        start_gather(b, b)
    wait_gather(0, 0)
    start_store(0, 0)
    wait_gather(1, 1)
    start_store(1, 1)

    @pl.loop(1, N_OUTER)
    def _body(g):
        for b in range(NBUF):
            i = g * NBUF + b
            wait_store(i - NBUF, b)
            start_gather(i, b)
            bj = (b + 2) % NBUF
            wait_gather(i - 2, bj)
            start_store(i - 2, bj)

    n = N_CHUNKS
    wait_gather(n - 2, (n - 2) % NBUF)
    start_store(n - 2, (n - 2) % NBUF)
    wait_gather(n - 1, (n - 1) % NBUF)
    start_store(n - 1, (n - 1) % NBUF)
    for k in range(NBUF):
        i = n - NBUF + k
        wait_store(i, i % NBUF)


def kernel(tokens, table):
    table_p = jnp.pad(table, ((0, 0), (0, PDIM - DIM)))
    out_p = _gather(tokens, table_p)
    return out_p[:, :DIM] * jnp.float32(1.0)


# speedup vs baseline: 1.0053x; 1.0053x over previous
import functools

import jax
import jax.numpy as jnp
from jax import lax
from jax.experimental import pallas as pl
from jax.experimental.pallas import tpu as pltpu
from jax.experimental.pallas import tpu_sc as plsc

DIM = 64
PDIM = 128
BATCH = 16384
HIST = 50
B = BATCH * HIST

_info = plsc.get_sparse_core_info()
NC = _info.num_cores
NS = _info.num_subcores
NW = NC * NS
ROWS_PER_W = BATCH // NW
B_PER_W = ROWS_PER_W * HIST
CHUNK = 128
N_CHUNKS = B_PER_W // CHUNK
NBUF = 4
N_OUTER = N_CHUNKS // NBUF
L = 16

_mesh = plsc.VectorSubcoreMesh(core_axis_name="c", subcore_axis_name="s")


@functools.partial(
    pl.kernel,
    mesh=_mesh,
    out_type=jax.ShapeDtypeStruct((B, PDIM), jnp.float32),
    scratch_types=[
        pltpu.VMEM((ROWS_PER_W, HIST), jnp.int32),
        pltpu.VMEM((B_PER_W,), jnp.int32),
        *[pltpu.VMEM((CHUNK, PDIM), jnp.float32) for _ in range(NBUF)],
        *[pltpu.SemaphoreType.DMA for _ in range(2 * NBUF)],
    ],
    compiler_params=pltpu.CompilerParams(use_tc_tiling_on_sc=False),
)
def _gather(idx_hbm, table_hbm, out_hbm, idx2d, idx_flat,
            r0, r1, r2, r3, g0, g1, g2, g3, s0, s1, s2, s3):
    rows = (r0, r1, r2, r3)
    semG = (g0, g1, g2, g3)
    semS = (s0, s1, s2, s3)
    wid = lax.axis_index("s") * NC + lax.axis_index("c")
    base = wid * B_PER_W

    pltpu.sync_copy(idx_hbm.at[pl.ds(wid * ROWS_PER_W, ROWS_PER_W)], idx2d)

    @pl.loop(0, ROWS_PER_W, unroll=8)
    def _flatten(r):
        off = r * HIST
        idx_flat[pl.ds(off, L)] = idx2d[r, pl.ds(0, L)]
        idx_flat[pl.ds(off + L, L)] = idx2d[r, pl.ds(L, L)]
        idx_flat[pl.ds(off + 2 * L, L)] = idx2d[r, pl.ds(2 * L, L)]
        idx_flat[pl.ds(off + HIST - L, L)] = idx2d[r, pl.ds(HIST - L, L)]

    def start_gather(i, b):
        pltpu.async_copy(
            table_hbm.at[idx_flat.at[pl.ds(i * CHUNK, CHUNK)]], rows[b], semG[b])

    def wait_gather(i, b):
        pltpu.make_async_copy(
            table_hbm.at[idx_flat.at[pl.ds(i * CHUNK, CHUNK)]], rows[b], semG[b]).wait()

    def start_store(i, b):
        pltpu.async_copy(
            rows[b].at[:, pl.ds(0, DIM)],
            out_hbm.at[pl.ds(base + i * CHUNK, CHUNK), pl.ds(0, DIM)], semS[b])

    def wait_store(i, b):
        pltpu.make_async_copy(
            rows[b].at[:, pl.ds(0, DIM)],
            out_hbm.at[pl.ds(base + i * CHUNK, CHUNK), pl.ds(0, DIM)], semS[b]).wait()

    for b in range(NBUF):
        start_gather(b, b)
    wait_gather(0, 0)
    start_store(0, 0)
    wait_gather(1, 1)
    start_store(1, 1)

    @pl.loop(1, N_OUTER)
    def _body(g):
        for b in range(NBUF):
            i = g * NBUF + b
            wait_store(i - NBUF, b)
            start_gather(i, b)
            bj = (b + 2) % NBUF
            wait_gather(i - 2, bj)
            start_store(i - 2, bj)

    n = N_CHUNKS
    wait_gather(n - 2, (n - 2) % NBUF)
    start_store(n - 2, (n - 2) % NBUF)
    wait_gather(n - 1, (n - 1) % NBUF)
    start_store(n - 1, (n - 1) % NBUF)
    for k in range(NBUF):
        i = n - NBUF + k
        wait_store(i, i % NBUF)


def kernel(tokens, table):
    table_p = jnp.pad(table, ((0, 0), (0, PDIM - DIM)))
    out_p = _gather(tokens, table_p)
    return out_p[:, :DIM]
